# baseline (speedup 1.0000x reference)
import jax
import jax.numpy as jnp
import numpy as np
from jax.experimental import pallas as pl
from jax.experimental.pallas import tpu as pltpu
from jax.sharding import Mesh, PartitionSpec as P


def _body(x_ref, a_ref, w_ref, out_ref, h_ref):
    @pl.when(pl.program_id(0) == 0)
    def _():
        h_ref[...] = jnp.dot(
            x_ref[...], w_ref[...], preferred_element_type=jnp.float32
        )

    out_ref[...] = jnp.dot(
        a_ref[...], h_ref[...], preferred_element_type=jnp.float32
    )


def _fused_gcn(x, A, W):
    M = A.shape[0]
    N, d_in = x.shape
    d_out = W.shape[1]

    bm = 8
    for cand in range(8, min(M, 2048) + 1, 8):
        if M % cand == 0 and cand * N * 4 * 2 <= 26 * 1024 * 1024:
            bm = cand

    return pl.pallas_call(
        _body,
        grid=(M // bm,),
        in_specs=[
            pl.BlockSpec((N, d_in), lambda i: (0, 0)),
            pl.BlockSpec((bm, N), lambda i: (i, 0)),
            pl.BlockSpec((d_in, d_out), lambda i: (0, 0)),
        ],
        out_specs=pl.BlockSpec((bm, d_out), lambda i: (i, 0)),
        out_shape=jax.ShapeDtypeStruct((M, d_out), jnp.float32),
        scratch_shapes=[pltpu.VMEM((N, d_out), jnp.float32)],
    )(x, A, W)


def kernel(x, A, W):
    N = A.shape[0]
    devs = jax.devices()
    n_shards = 1
    for cand in (len(devs), 2):
        if cand > 1 and N % cand == 0:
            n_shards = cand
            break
    if n_shards == 1:
        return _fused_gcn(x, A, W)

    mesh = Mesh(np.array(devs[:n_shards]), ("i",))
    sharded = jax.shard_map(
        _fused_gcn,
        mesh=mesh,
        in_specs=(P(None, None), P("i", None), P(None, None)),
        out_specs=P("i", None),
        check_vma=False,
    )
    return sharded(x, A, W)


# speedup vs baseline: 5.3160x; 5.3160x over previous
import jax
import jax.numpy as jnp
from jax.experimental import pallas as pl
from jax.experimental.pallas import tpu as pltpu


def _body(x_ref, a0_ref, a1_ref, w_ref, out_ref, h_ref):
    @pl.when(pl.program_id(0) == 0)
    def _():
        h_ref[...] = jnp.dot(
            x_ref[...], w_ref[...], preferred_element_type=jnp.float32
        )

    bm = a0_ref.shape[0]
    out_ref[:bm, :] = jnp.dot(
        a0_ref[...], h_ref[...], preferred_element_type=jnp.float32
    )
    out_ref[bm:, :] = jnp.dot(
        a1_ref[...], h_ref[...], preferred_element_type=jnp.float32
    )


def kernel(x, A, W):
    N, d_in = x.shape
    d_out = W.shape[1]

    bm = 8
    for cand in range(8, min(N, 2048) + 1, 8):
        if N % (2 * cand) == 0 and cand * N * 4 * 4 <= 40 * 1024 * 1024:
            bm = cand

    return pl.pallas_call(
        _body,
        grid=(N // (2 * bm),),
        in_specs=[
            pl.BlockSpec((N, d_in), lambda i: (0, 0)),
            pl.BlockSpec((bm, N), lambda i: (2 * i, 0)),
            pl.BlockSpec((bm, N), lambda i: (2 * i + 1, 0)),
            pl.BlockSpec((d_in, d_out), lambda i: (0, 0)),
        ],
        out_specs=pl.BlockSpec((2 * bm, d_out), lambda i: (i, 0)),
        out_shape=jax.ShapeDtypeStruct((N, d_out), jnp.float32),
        scratch_shapes=[pltpu.VMEM((N, d_out), jnp.float32)],
    )(x, A, A, W)
